# Initial kernel scaffold; baseline (speedup 1.0000x reference)
import jax
import jax.numpy as jnp
from jax.experimental import pallas as pl


def kernel(pred_boxes, id, parent_id, type_id):
    raise NotImplementedError("write your pallas kernel here")


# speedup vs baseline: 1.5794x; 1.5794x over previous
import jax
import jax.numpy as jnp
from jax import lax
from jax.experimental import pallas as pl
from jax.experimental.pallas import tpu as pltpu

_B = 64
_S = 512
_M = _S // 2
_IMG_W = 1440.0
_IMG_H = 2560.0


def _pair_body(idv_r, idv_c, pidv_r, pidv_c,
               cx_r, cy_r, w_r, h_r, cx_c, cy_c, w_c, h_c,
               tot_ref, cnt_ref):
    b = pl.program_id(0)

    @pl.when(b == 0)
    def _():
        tot_ref[...] = jnp.zeros((1, 1), jnp.float32)
        cnt_ref[...] = jnp.zeros((1, 1), jnp.float32)

    M = _M
    row = lax.broadcasted_iota(jnp.int32, (M, M), 0)
    col = lax.broadcasted_iota(jnp.int32, (M, M), 1)

    ir = idv_r[0]
    ic = idv_c[0]
    P = ic == ir

    dup_c = jnp.any(P & (col > row), axis=1, keepdims=True)
    dup_r = jnp.any(P & (row > col), axis=0, keepdims=True)
    mask = (~dup_c) & (~dup_r) & (pidv_c[0] == pidv_r[0]) & (row < col)

    def coords(cx, cy, w, h):
        cxs = cx * _IMG_W
        cys = cy * _IMG_H
        ws = w * _IMG_W
        hs = h * _IMG_H
        return (cxs - ws * 0.5, cys - hs * 0.5,
                cxs + ws * 0.5, cys + hs * 0.5, ws * hs)

    x1r, y1r, x2r, y2r, ar = coords(cx_r[0], cy_r[0], w_r[0], h_r[0])
    x1c, y1c, x2c, y2c, ac = coords(cx_c[0], cy_c[0], w_c[0], h_c[0])

    xl = jnp.maximum(x1c, x1r)
    yt = jnp.maximum(y1c, y1r)
    xr = jnp.minimum(x2c, x2r)
    yb = jnp.minimum(y2c, y2r)
    legal = (xr >= xl) & (yb >= yt) & mask
    inter = (xr - xl) * (yb - yt)
    amin = jnp.minimum(ac, ar)
    iou = jnp.where(legal, inter / amin, 0.0)

    tot_ref[...] += jnp.sum(iou).reshape(1, 1)
    cnt_ref[...] += jnp.sum(legal.astype(jnp.float32)).reshape(1, 1)


def _run_pair_kernel(idv_r, idv_c, pidv_r, pidv_c, comps_r, comps_c,
                     interpret=False):
    B, M = _B, _M
    row_spec = pl.BlockSpec((1, 1, M), lambda b: (b, 0, 0))
    col_spec = pl.BlockSpec((1, M, 1), lambda b: (b, 0, 0))
    out_spec = pl.BlockSpec((1, 1), lambda b: (0, 0))
    grid = (B,)
    tot, cnt = pl.pallas_call(
        _pair_body,
        grid=grid,
        in_specs=[row_spec, col_spec, row_spec, col_spec,
                  row_spec, row_spec, row_spec, row_spec,
                  col_spec, col_spec, col_spec, col_spec],
        out_specs=[out_spec, out_spec],
        out_shape=[jax.ShapeDtypeStruct((1, 1), jnp.float32),
                   jax.ShapeDtypeStruct((1, 1), jnp.float32)],
        compiler_params=pltpu.CompilerParams(
            dimension_semantics=("arbitrary",)),
        interpret=interpret,
    )(idv_r, idv_c, pidv_r, pidv_c, *comps_r, *comps_c)
    return tot[0, 0], cnt[0, 0]


def kernel(pred_boxes, id, parent_id, type_id):
    B, M = _B, _M
    idv = id[:, 1::2].astype(jnp.int32)
    pidv = parent_id[:, 1::2].astype(jnp.int32)
    pb = pred_boxes[:, 1::2, :]

    idv_r = idv.reshape(B, 1, M)
    idv_c = idv.reshape(B, M, 1)
    pidv_r = pidv.reshape(B, 1, M)
    pidv_c = pidv.reshape(B, M, 1)
    comps_r = [pb[:, :, k].reshape(B, 1, M) for k in range(4)]
    comps_c = [pb[:, :, k].reshape(B, M, 1) for k in range(4)]

    total, cnt = _run_pair_kernel(idv_r, idv_c, pidv_r, pidv_c,
                                  comps_r, comps_c)
    ratio = total / cnt
    bad = (cnt == 0) | jnp.logical_not(ratio >= 0.0) | jnp.logical_not(ratio <= 1.0)
    return jnp.where(bad, jnp.asarray(0.0, dtype=jnp.float32), total)


# speedup vs baseline: 2.9577x; 1.8727x over previous
import functools

import jax
import jax.numpy as jnp
from jax import lax
from jax.experimental import pallas as pl
from jax.experimental.pallas import tpu as pltpu
from jax.experimental.pallas import tpu_sc as plsc

_B = 64
_S = 512
_M = _S // 2
_IMG_W = 1440.0
_IMG_H = 2560.0
_NW = 32
_CAP = _M + 16


def _sc_body(ids_hbm, flt_hbm, out_hbm,
             idv_v, pidv_v, cx_v, cy_v, w_v, h_v,
             x1_v, y1_v, x2_v, y2_v, ar_v, last_v,
             gx1, gy1, gx2, gy2, gar, gpid, accrow_v, sem):
    cid = lax.axis_index("c")
    sid = lax.axis_index("s")
    wid = sid * 2 + cid
    iota = lax.iota(jnp.int32, 16)

    tot = jnp.zeros((16,), jnp.float32)
    cnt = jnp.zeros((16,), jnp.float32)
    for bi in range(2):
        b = wid * 2 + bi
        cps = [
            pltpu.async_copy(ids_hbm.at[b, 0], idv_v, sem),
            pltpu.async_copy(ids_hbm.at[b, 1], pidv_v, sem),
            pltpu.async_copy(flt_hbm.at[b, 0], cx_v, sem),
            pltpu.async_copy(flt_hbm.at[b, 1], cy_v, sem),
            pltpu.async_copy(flt_hbm.at[b, 2], w_v, sem),
            pltpu.async_copy(flt_hbm.at[b, 3], h_v, sem),
        ]
        for cp in cps:
            cp.wait()

        def chunk_b(c, carry):
            s = pl.ds(c * 16, 16)
            cx = cx_v[s] * _IMG_W
            cy = cy_v[s] * _IMG_H
            wp = w_v[s] * _IMG_W
            hp = h_v[s] * _IMG_H
            x1_v[s] = cx - wp * 0.5
            y1_v[s] = cy - hp * 0.5
            x2_v[s] = cx + wp * 0.5
            y2_v[s] = cy + hp * 0.5
            ar_v[s] = wp * hp
            v = idv_v[s]
            _, lastm = plsc.scan_count(v)
            plsc.store_scatter(last_v, [v], c * 16 + iota, mask=lastm)
            return carry

        lax.fori_loop(0, 16, chunk_b, jnp.int32(0))

        def chunk_c(c, off):
            s = pl.ds(c * 16, 16)
            v = idv_v[s]
            pos = c * 16 + iota
            rep = plsc.load_gather(last_v, [v]) == pos
            d = pl.ds(off, 16)
            plsc.store_compressed(gx1.at[d], x1_v[s], mask=rep)
            plsc.store_compressed(gy1.at[d], y1_v[s], mask=rep)
            plsc.store_compressed(gx2.at[d], x2_v[s], mask=rep)
            plsc.store_compressed(gy2.at[d], y2_v[s], mask=rep)
            plsc.store_compressed(gar.at[d], ar_v[s], mask=rep)
            plsc.store_compressed(gpid.at[d], pidv_v[s], mask=rep)
            return off + jnp.sum(rep.astype(jnp.int32))

        n = lax.fori_loop(0, 16, chunk_c, jnp.int32(0))
        nch = (n + 15) // 16

        def row(i, carry):
            ii = jnp.full((16,), i, jnp.int32)
            rx1 = plsc.load_gather(gx1, [ii])
            ry1 = plsc.load_gather(gy1, [ii])
            rx2 = plsc.load_gather(gx2, [ii])
            ry2 = plsc.load_gather(gy2, [ii])
            rar = plsc.load_gather(gar, [ii])
            rpid = plsc.load_gather(gpid, [ii])

            def col_chunk(c, carry2):
                tot2, cnt2 = carry2
                s = pl.ds(c * 16, 16)
                colidx = c * 16 + iota
                m = (colidx > i) & (colidx < n) & (gpid[s] == rpid)
                xl = jnp.maximum(rx1, gx1[s])
                yt = jnp.maximum(ry1, gy1[s])
                xr = jnp.minimum(rx2, gx2[s])
                yb = jnp.minimum(ry2, gy2[s])
                legal = (xr >= xl) & (yb >= yt) & m
                inter = (xr - xl) * (yb - yt)
                amin = jnp.minimum(rar, gar[s])
                iou = jnp.where(legal, inter / amin, jnp.float32(0.0))
                return tot2 + iou, cnt2 + legal.astype(jnp.float32)

            return lax.fori_loop(i // 16, nch, col_chunk, carry)

        tot, cnt = lax.fori_loop(0, n, row, (tot, cnt))

    accrow_v[...] = tot
    pltpu.sync_copy(accrow_v, out_hbm.at[wid])
    accrow_v[...] = cnt
    pltpu.sync_copy(accrow_v, out_hbm.at[_NW + wid])


@jax.jit
def _sc_call(ids_in, flt_in):
    mesh = plsc.VectorSubcoreMesh(core_axis_name="c", subcore_axis_name="s")
    run = pl.kernel(
        _sc_body,
        out_type=jax.ShapeDtypeStruct((2 * _NW, 16), jnp.float32),
        mesh=mesh,
        compiler_params=pltpu.CompilerParams(needs_layout_passes=False),
        scratch_types=[
            pltpu.VMEM((_M,), jnp.int32),
            pltpu.VMEM((_M,), jnp.int32),
            pltpu.VMEM((_M,), jnp.float32),
            pltpu.VMEM((_M,), jnp.float32),
            pltpu.VMEM((_M,), jnp.float32),
            pltpu.VMEM((_M,), jnp.float32),
            pltpu.VMEM((_M,), jnp.float32),
            pltpu.VMEM((_M,), jnp.float32),
            pltpu.VMEM((_M,), jnp.float32),
            pltpu.VMEM((_M,), jnp.float32),
            pltpu.VMEM((_M,), jnp.float32),
            pltpu.VMEM((_M,), jnp.int32),
            pltpu.VMEM((_CAP,), jnp.float32),
            pltpu.VMEM((_CAP,), jnp.float32),
            pltpu.VMEM((_CAP,), jnp.float32),
            pltpu.VMEM((_CAP,), jnp.float32),
            pltpu.VMEM((_CAP,), jnp.float32),
            pltpu.VMEM((_CAP,), jnp.int32),
            pltpu.VMEM((16,), jnp.float32),
            pltpu.SemaphoreType.DMA,
        ],
    )
    return run(ids_in, flt_in)


def kernel(pred_boxes, id, parent_id, type_id):
    B, M = _B, _M
    idv = id[:, 1::2].astype(jnp.int32)
    pidv = parent_id[:, 1::2].astype(jnp.int32)
    pb = pred_boxes[:, 1::2, :]

    ids_in = jnp.stack([idv, pidv], axis=1)
    flt_in = jnp.transpose(pb, (0, 2, 1))

    out = _sc_call(ids_in, flt_in)
    total = jnp.sum(out[:_NW])
    cntf = jnp.sum(out[_NW:])
    ratio = total / cntf
    bad = (cntf == 0) | jnp.logical_not(ratio >= 0.0) | jnp.logical_not(ratio <= 1.0)
    return jnp.where(bad, jnp.asarray(0.0, dtype=jnp.float32), total)


# speedup vs baseline: 3.0662x; 1.0367x over previous
import jax
import jax.numpy as jnp
from jax import lax
from jax.experimental import pallas as pl
from jax.experimental.pallas import tpu as pltpu
from jax.experimental.pallas import tpu_sc as plsc

_B = 64
_S = 512
_M = _S // 2
_IMG_W = 1440.0
_IMG_H = 2560.0
_NW = 32
_CAP = _M + 16


def _sc_body(id_hbm, pid_hbm, box_hbm, out_hbm,
             idr_v, pidr_v, box_v,
             idv_v, pidv_v,
             x1_v, y1_v, x2_v, y2_v, ar_v, last_v,
             gx1, gy1, gx2, gy2, gar, gpid, accrow_v, sem):
    cid = lax.axis_index("c")
    sid = lax.axis_index("s")
    wid = sid * 2 + cid
    iota = lax.iota(jnp.int32, 16)
    zeros16 = jnp.zeros((16,), jnp.int32)

    tot = jnp.zeros((16,), jnp.float32)
    cnt = jnp.zeros((16,), jnp.float32)
    for bi in range(2):
        b = wid * 2 + bi
        cps = [
            pltpu.async_copy(id_hbm.at[b], idr_v, sem),
            pltpu.async_copy(pid_hbm.at[b], pidr_v, sem),
            pltpu.async_copy(box_hbm.at[b], box_v, sem),
        ]
        for cp in cps:
            cp.wait()

        def chunk_b(c, carry):
            s = pl.ds(c * 16, 16)
            pos = c * 16 + iota
            oidx = 2 * pos + 1
            v = plsc.load_gather(idr_v, [oidx])
            p = plsc.load_gather(pidr_v, [oidx])
            idv_v[s] = v
            pidv_v[s] = p
            cx = plsc.load_gather(box_v, [oidx, zeros16]) * _IMG_W
            cy = plsc.load_gather(box_v, [oidx, zeros16 + 1]) * _IMG_H
            wp = plsc.load_gather(box_v, [oidx, zeros16 + 2]) * _IMG_W
            hp = plsc.load_gather(box_v, [oidx, zeros16 + 3]) * _IMG_H
            x1_v[s] = cx - wp * 0.5
            y1_v[s] = cy - hp * 0.5
            x2_v[s] = cx + wp * 0.5
            y2_v[s] = cy + hp * 0.5
            ar_v[s] = wp * hp
            _, lastm = plsc.scan_count(v)
            plsc.store_scatter(last_v, [v], pos, mask=lastm)
            return carry

        lax.fori_loop(0, 16, chunk_b, jnp.int32(0))

        def chunk_c(c, off):
            s = pl.ds(c * 16, 16)
            v = idv_v[s]
            pos = c * 16 + iota
            rep = plsc.load_gather(last_v, [v]) == pos
            d = pl.ds(off, 16)
            plsc.store_compressed(gx1.at[d], x1_v[s], mask=rep)
            plsc.store_compressed(gy1.at[d], y1_v[s], mask=rep)
            plsc.store_compressed(gx2.at[d], x2_v[s], mask=rep)
            plsc.store_compressed(gy2.at[d], y2_v[s], mask=rep)
            plsc.store_compressed(gar.at[d], ar_v[s], mask=rep)
            plsc.store_compressed(gpid.at[d], pidv_v[s], mask=rep)
            return off + jnp.sum(rep.astype(jnp.int32))

        n = lax.fori_loop(0, 16, chunk_c, jnp.int32(0))
        nch = (n + 15) // 16

        def row(i, carry):
            ii = jnp.full((16,), i, jnp.int32)
            rx1 = plsc.load_gather(gx1, [ii])
            ry1 = plsc.load_gather(gy1, [ii])
            rx2 = plsc.load_gather(gx2, [ii])
            ry2 = plsc.load_gather(gy2, [ii])
            rar = plsc.load_gather(gar, [ii])
            rpid = plsc.load_gather(gpid, [ii])

            def col_chunk(c, carry2):
                tot2, cnt2 = carry2
                s = pl.ds(c * 16, 16)
                colidx = c * 16 + iota
                m = (colidx > i) & (colidx < n) & (gpid[s] == rpid)
                xl = jnp.maximum(rx1, gx1[s])
                yt = jnp.maximum(ry1, gy1[s])
                xr = jnp.minimum(rx2, gx2[s])
                yb = jnp.minimum(ry2, gy2[s])
                legal = (xr >= xl) & (yb >= yt) & m
                inter = (xr - xl) * (yb - yt)
                amin = jnp.minimum(rar, gar[s])
                iou = jnp.where(legal, inter / amin, jnp.float32(0.0))
                return tot2 + iou, cnt2 + legal.astype(jnp.float32)

            return lax.fori_loop(i // 16, nch, col_chunk, carry)

        tot, cnt = lax.fori_loop(0, n, row, (tot, cnt))

    accrow_v[...] = tot
    pltpu.sync_copy(accrow_v, out_hbm.at[wid])
    accrow_v[...] = cnt
    pltpu.sync_copy(accrow_v, out_hbm.at[_NW + wid])


@jax.jit
def _sc_call(id_in, pid_in, box_in):
    mesh = plsc.VectorSubcoreMesh(core_axis_name="c", subcore_axis_name="s")
    run = pl.kernel(
        _sc_body,
        out_type=jax.ShapeDtypeStruct((2 * _NW, 16), jnp.float32),
        mesh=mesh,
        compiler_params=pltpu.CompilerParams(needs_layout_passes=False),
        scratch_types=[
            pltpu.VMEM((_S,), jnp.int32),
            pltpu.VMEM((_S,), jnp.int32),
            pltpu.VMEM((_S, 4), jnp.float32),
            pltpu.VMEM((_M,), jnp.int32),
            pltpu.VMEM((_M,), jnp.int32),
            pltpu.VMEM((_M,), jnp.float32),
            pltpu.VMEM((_M,), jnp.float32),
            pltpu.VMEM((_M,), jnp.float32),
            pltpu.VMEM((_M,), jnp.float32),
            pltpu.VMEM((_M,), jnp.float32),
            pltpu.VMEM((_M,), jnp.int32),
            pltpu.VMEM((_CAP,), jnp.float32),
            pltpu.VMEM((_CAP,), jnp.float32),
            pltpu.VMEM((_CAP,), jnp.float32),
            pltpu.VMEM((_CAP,), jnp.float32),
            pltpu.VMEM((_CAP,), jnp.float32),
            pltpu.VMEM((_CAP,), jnp.int32),
            pltpu.VMEM((16,), jnp.float32),
            pltpu.SemaphoreType.DMA,
        ],
    )
    return run(id_in, pid_in, box_in)


def kernel(pred_boxes, id, parent_id, type_id):
    out = _sc_call(id.astype(jnp.int32), parent_id.astype(jnp.int32),
                   pred_boxes)
    total = jnp.sum(out[:_NW])
    cntf = jnp.sum(out[_NW:])
    ratio = total / cntf
    bad = (cntf == 0) | jnp.logical_not(ratio >= 0.0) | jnp.logical_not(ratio <= 1.0)
    return jnp.where(bad, jnp.asarray(0.0, dtype=jnp.float32), total)
